# baseline (speedup 1.0000x reference)
import jax
import jax.numpy as jnp
from jax import lax
from jax.experimental import pallas as pl
from jax.experimental.pallas import tpu as pltpu
from jax.experimental.pallas import tpu_sc as plsc

NC = 2
NS = 16
NW = NC * NS
KM = 56
NB = 4

_MESH = plsc.VectorSubcoreMesh(core_axis_name="c", subcore_axis_name="s")
_SC_UNTILED = pltpu.CompilerParams(use_tc_tiling_on_sc=False)


def _sc_degree(dst3, ones_rows, zeros16, npad):
    nchunks, kd = dst3.shape[1:]
    rps = npad // NS

    def body(dst_hbm, ones_hbm, zeros_hbm, out_hbm, dst_v, ones_v, acc):
        cid = lax.axis_index("c")
        sid = lax.axis_index("s")
        wid = sid * NC + cid
        pltpu.sync_copy(zeros_hbm.at[pl.ds(sid * rps, rps)],
                        acc.at[pl.ds(sid * rps, rps)])
        pltpu.sync_copy(ones_hbm, ones_v)
        pltpu.sync_copy(dst_hbm.at[wid], dst_v)
        plsc.subcore_barrier()

        def step(c, carry):
            pltpu.sync_copy(ones_v, acc.at[dst_v.at[c]], add=True)
            return carry

        lax.fori_loop(0, nchunks, step, 0)
        plsc.subcore_barrier()
        pltpu.sync_copy(acc.at[pl.ds(sid * rps, rps)],
                        out_hbm.at[cid, pl.ds(sid * rps, rps)])

    f = pl.kernel(
        body,
        out_type=jax.ShapeDtypeStruct((NC, npad, 16), jnp.float32),
        mesh=_MESH,
        compiler_params=_SC_UNTILED,
        scratch_types=[
            pltpu.VMEM((nchunks, kd), jnp.int32),
            pltpu.VMEM((kd, 16), jnp.float32),
            pltpu.VMEM_SHARED((npad, 16), jnp.float32),
        ],
    )
    return f(dst3, ones_rows, zeros16)


def _sc_messages(g, src3, dst3, zeros, npad):
    n, d = g.shape
    nchunks = src3.shape[1]
    rps = npad // NS

    def body(g_hbm, src_hbm, dst_hbm, zeros_hbm, out_hbm,
             src_v, dst_v, rows, gsems, ssems, acc):
        cid = lax.axis_index("c")
        sid = lax.axis_index("s")
        wid = sid * NC + cid
        pltpu.sync_copy(zeros_hbm.at[pl.ds(sid * rps, rps)],
                        acc.at[pl.ds(sid * rps, rps)])
        pltpu.sync_copy(src_hbm.at[wid], src_v)
        pltpu.sync_copy(dst_hbm.at[wid], dst_v)
        plsc.subcore_barrier()

        def gather(c, b):
            pltpu.async_copy(g_hbm.at[src_v.at[c]], rows.at[b], gsems[b])

        def wait_gather(c, b):
            pltpu.make_async_copy(g_hbm.at[src_v.at[c]], rows.at[b],
                                  gsems[b]).wait()

        def scatter(c, b):
            pltpu.async_copy(rows.at[b], acc.at[dst_v.at[c]], ssems[b],
                             add=True)

        def wait_scatter(c, b):
            pltpu.make_async_copy(rows.at[b], acc.at[dst_v.at[c]],
                                  ssems[b]).wait()

        gather(0, 0)
        gather(1, 1)

        def step(c4, carry):
            for j in range(NB):
                c = NB * c4 + j
                b2 = (j + 2) % NB

                @pl.when(c + 2 < nchunks)
                def _():
                    @pl.when(c >= 2)
                    def _():
                        wait_scatter(c - 2, b2)
                    gather(c + 2, b2)

                wait_gather(c, j)
                scatter(c, j)
            return carry

        lax.fori_loop(0, nchunks // NB, step, 0)
        for j in range(NB):
            wait_scatter(nchunks - NB + j, j)
        plsc.subcore_barrier()
        pltpu.sync_copy(acc.at[pl.ds(sid * rps, rps)],
                        out_hbm.at[cid, pl.ds(sid * rps, rps)])

    f = pl.kernel(
        body,
        out_type=jax.ShapeDtypeStruct((NC, npad, d), jnp.float32),
        mesh=_MESH,
        compiler_params=_SC_UNTILED,
        scratch_types=[
            pltpu.VMEM((nchunks, KM), jnp.int32),
            pltpu.VMEM((nchunks, KM), jnp.int32),
            pltpu.VMEM((NB, KM, d), jnp.float32),
            [pltpu.SemaphoreType.DMA] * NB,
            [pltpu.SemaphoreType.DMA] * NB,
            pltpu.VMEM_SHARED((npad, d), jnp.float32),
        ],
    )
    return f(g, src3, dst3, zeros)


_ROWS = 1000


def _deg_specs():
    return [
        pl.BlockSpec((1, _ROWS, 16), lambda i: (0, i, 0)),
        pl.BlockSpec((1, _ROWS, 16), lambda i: (1, i, 0)),
    ]


def _sp_specs(dh):
    return [
        pl.BlockSpec((1, _ROWS, dh), lambda i: (0, i, 0)),
        pl.BlockSpec((1, _ROWS, dh), lambda i: (1, i, 0)),
    ]


def _tc_scale_matmul(x, w, degp):
    n, din = x.shape
    dh = w.shape[1]

    def body(x_ref, w_ref, dp0_ref, dp1_ref, g_ref):
        dinv = lax.rsqrt(dp0_ref[0][:, :1] + dp1_ref[0][:, :1] + 1.0)
        h = jnp.dot(x_ref[...], w_ref[...], preferred_element_type=jnp.float32)
        g_ref[...] = h * dinv

    return pl.pallas_call(
        body,
        grid=(n // _ROWS,),
        in_specs=[
            pl.BlockSpec((_ROWS, din), lambda i: (i, 0)),
            pl.BlockSpec((din, dh), lambda i: (0, 0)),
        ] + _deg_specs(),
        out_specs=pl.BlockSpec((_ROWS, dh), lambda i: (i, 0)),
        out_shape=jax.ShapeDtypeStruct((n, dh), jnp.float32),
    )(x, w, degp, degp)


def _tc_combine_matmul(sp, g, degp, b, w):
    n, dh = g.shape
    dout = w.shape[1]

    def body(s0_ref, s1_ref, g_ref, dp0_ref, dp1_ref, b_ref, w_ref, o_ref):
        dinv = lax.rsqrt(dp0_ref[0][:, :1] + dp1_ref[0][:, :1] + 1.0)
        s = s0_ref[0] + s1_ref[0] + g_ref[...]
        x2 = jnp.maximum(dinv * s + b_ref[...], 0.0)
        h2 = jnp.dot(x2, w_ref[...], preferred_element_type=jnp.float32)
        o_ref[...] = h2 * dinv

    return pl.pallas_call(
        body,
        grid=(n // _ROWS,),
        in_specs=_sp_specs(dh) + [
            pl.BlockSpec((_ROWS, dh), lambda i: (i, 0)),
        ] + _deg_specs() + [
            pl.BlockSpec((1, dh), lambda i: (0, 0)),
            pl.BlockSpec((dh, dout), lambda i: (0, 0)),
        ],
        out_specs=pl.BlockSpec((_ROWS, dout), lambda i: (i, 0)),
        out_shape=jax.ShapeDtypeStruct((n, dout), jnp.float32),
    )(sp, sp, g, degp, degp, b, w)


def _tc_combine(sp, g, degp, b):
    n, dh = g.shape

    def body(s0_ref, s1_ref, g_ref, dp0_ref, dp1_ref, b_ref, o_ref):
        dinv = lax.rsqrt(dp0_ref[0][:, :1] + dp1_ref[0][:, :1] + 1.0)
        s = s0_ref[0] + s1_ref[0] + g_ref[...]
        o_ref[...] = jnp.maximum(dinv * s + b_ref[...], 0.0)

    return pl.pallas_call(
        body,
        grid=(n // _ROWS,),
        in_specs=_sp_specs(dh) + [
            pl.BlockSpec((_ROWS, dh), lambda i: (i, 0)),
        ] + _deg_specs() + [
            pl.BlockSpec((1, dh), lambda i: (0, 0)),
        ],
        out_specs=pl.BlockSpec((_ROWS, dh), lambda i: (i, 0)),
        out_shape=jax.ShapeDtypeStruct((n, dh), jnp.float32),
    )(sp, sp, g, degp, degp, b)


def kernel(x, edge_index, W1, b1, W2, b2):
    n, _ = x.shape
    e = edge_index.shape[1]
    dh = W1.shape[1]
    rps = -(-((n + 1) // NS + 7) // 8) * 8
    npad = NS * rps

    c_msg = -(-(-(-e // (NW * KM))) // NB) * NB
    epad = NW * c_msg * KM - e
    pad_i = jnp.arange(epad, dtype=jnp.int32)
    src3 = jnp.concatenate(
        [edge_index[0], pad_i % n]).reshape(NW, c_msg, KM)
    dst3 = jnp.concatenate(
        [edge_index[1], n + pad_i % (npad - n)]).reshape(NW, c_msg, KM)

    kd_opts = [k for k in range(8, 129, 8) if e % (NW * k) == 0]
    if kd_opts:
        kd = max(kd_opts)
        dst3d = edge_index[1].reshape(NW, e // (NW * kd), kd)
    else:
        epw = c_msg * KM
        kd = max(k for k in range(8, 129, 8) if epw % k == 0)
        dst3d = dst3.reshape(NW, epw // kd, kd)

    ones_rows = jnp.ones((kd, 16), jnp.float32)
    zeros16 = jnp.zeros((npad, 16), jnp.float32)
    zerosN = jnp.zeros((npad, dh), jnp.float32)
    b1r = b1.reshape(1, -1)
    b2r = b2.reshape(1, -1)

    degp = _sc_degree(dst3d, ones_rows, zeros16, npad)

    g1 = _tc_scale_matmul(x, W1, degp)
    sp1 = _sc_messages(g1, src3, dst3, zerosN, npad)
    g2 = _tc_combine_matmul(sp1, g1, degp, b1r, W2)
    sp2 = _sc_messages(g2, src3, dst3, zerosN, npad)
    return _tc_combine(sp2, g2, degp, b2r)


# speedup vs baseline: 1.0090x; 1.0090x over previous
import jax
import jax.numpy as jnp
from jax import lax
from jax.experimental import pallas as pl
from jax.experimental.pallas import tpu as pltpu
from jax.experimental.pallas import tpu_sc as plsc

NC = 2
NS = 16
NW = NC * NS
KM = 56
NB = 4

_MESH = plsc.VectorSubcoreMesh(core_axis_name="c", subcore_axis_name="s")
_SC_UNTILED = pltpu.CompilerParams(use_tc_tiling_on_sc=False)


def _sc_degree(dst3, ones_rows, zeros16, npad):
    nchunks, kd = dst3.shape[1:]
    rps = npad // NS

    def body(dst_hbm, ones_hbm, zeros_hbm, out_hbm, dst_v, ones_v, acc):
        cid = lax.axis_index("c")
        sid = lax.axis_index("s")
        wid = sid * NC + cid
        pltpu.sync_copy(zeros_hbm.at[pl.ds(sid * rps, rps)],
                        acc.at[pl.ds(sid * rps, rps)])
        pltpu.sync_copy(ones_hbm, ones_v)
        pltpu.sync_copy(dst_hbm.at[wid], dst_v)
        plsc.subcore_barrier()

        def step(c, carry):
            pltpu.sync_copy(ones_v, acc.at[dst_v.at[c]], add=True)
            return carry

        lax.fori_loop(0, nchunks, step, 0)
        plsc.subcore_barrier()
        pltpu.sync_copy(acc.at[pl.ds(sid * rps, rps)],
                        out_hbm.at[cid, pl.ds(sid * rps, rps)])

    f = pl.kernel(
        body,
        out_type=jax.ShapeDtypeStruct((NC, npad, 16), jnp.float32),
        mesh=_MESH,
        compiler_params=_SC_UNTILED,
        scratch_types=[
            pltpu.VMEM((nchunks, kd), jnp.int32),
            pltpu.VMEM((kd, 16), jnp.float32),
            pltpu.VMEM_SHARED((npad, 16), jnp.float32),
        ],
    )
    return f(dst3, ones_rows, zeros16)


def _sc_messages(g, src3, dst3, zeros, npad):
    n, d = g.shape
    nchunks = src3.shape[1]
    rps = npad // NS

    def body(g_hbm, src_hbm, dst_hbm, zeros_hbm, out_hbm,
             src_v, dst_v, rows, gsems, ssems, acc):
        cid = lax.axis_index("c")
        sid = lax.axis_index("s")
        wid = sid * NC + cid
        pltpu.sync_copy(zeros_hbm.at[pl.ds(sid * rps, rps)],
                        acc.at[pl.ds(sid * rps, rps)])
        pltpu.sync_copy(src_hbm.at[wid], src_v)
        pltpu.sync_copy(dst_hbm.at[wid], dst_v)
        plsc.subcore_barrier()

        def gather(c, b):
            pltpu.async_copy(g_hbm.at[src_v.at[c]], rows.at[b], gsems[b])

        def wait_gather(c, b):
            pltpu.make_async_copy(g_hbm.at[src_v.at[c]], rows.at[b],
                                  gsems[b]).wait()

        def scatter(c, b):
            pltpu.async_copy(rows.at[b], acc.at[dst_v.at[c]], ssems[b],
                             add=True)

        def wait_scatter(c, b):
            pltpu.make_async_copy(rows.at[b], acc.at[dst_v.at[c]],
                                  ssems[b]).wait()

        gather(0, 0)
        gather(1, 1)

        def step(c4, carry):
            for j in range(NB):
                c = NB * c4 + j
                b2 = (j + 2) % NB

                @pl.when(c + 2 < nchunks)
                def _():
                    @pl.when(c >= 2)
                    def _():
                        wait_scatter(c - 2, b2)
                    gather(c + 2, b2)

                wait_gather(c, j)
                scatter(c, j)
            return carry

        lax.fori_loop(0, nchunks // NB, step, 0)
        for j in range(NB):
            wait_scatter(nchunks - NB + j, j)
        plsc.subcore_barrier()
        pltpu.sync_copy(acc.at[pl.ds(sid * rps, rps)],
                        out_hbm.at[cid, pl.ds(sid * rps, rps)])

    f = pl.kernel(
        body,
        out_type=jax.ShapeDtypeStruct((NC, npad, d), jnp.float32),
        mesh=_MESH,
        compiler_params=_SC_UNTILED,
        scratch_types=[
            pltpu.VMEM((nchunks, KM), jnp.int32),
            pltpu.VMEM((nchunks, KM), jnp.int32),
            pltpu.VMEM((NB, KM, d), jnp.float32),
            [pltpu.SemaphoreType.DMA] * NB,
            [pltpu.SemaphoreType.DMA] * NB,
            pltpu.VMEM_SHARED((npad, d), jnp.float32),
        ],
    )
    return f(g, src3, dst3, zeros)


_ROWS = 1000


def _deg_specs():
    return [
        pl.BlockSpec((1, _ROWS, 16), lambda i: (0, i, 0)),
        pl.BlockSpec((1, _ROWS, 16), lambda i: (1, i, 0)),
    ]


def _sp_specs(dh):
    return [
        pl.BlockSpec((1, _ROWS, dh), lambda i: (0, i, 0)),
        pl.BlockSpec((1, _ROWS, dh), lambda i: (1, i, 0)),
    ]


def _tc_scale_matmul(x, w, degp):
    n, din = x.shape
    dh = w.shape[1]

    def body(x_ref, w_ref, dp0_ref, dp1_ref, g_ref):
        dinv = lax.rsqrt(dp0_ref[0][:, :1] + dp1_ref[0][:, :1] + 1.0)
        h = jnp.dot(x_ref[...], w_ref[...], preferred_element_type=jnp.float32)
        g_ref[...] = h * dinv

    return pl.pallas_call(
        body,
        grid=(n // _ROWS,),
        in_specs=[
            pl.BlockSpec((_ROWS, din), lambda i: (i, 0)),
            pl.BlockSpec((din, dh), lambda i: (0, 0)),
        ] + _deg_specs(),
        out_specs=pl.BlockSpec((_ROWS, dh), lambda i: (i, 0)),
        out_shape=jax.ShapeDtypeStruct((n, dh), jnp.float32),
    )(x, w, degp, degp)


def _tc_combine_matmul(sp, g, degp, b, w):
    n, dh = g.shape
    dout = w.shape[1]

    def body(s0_ref, s1_ref, g_ref, dp0_ref, dp1_ref, b_ref, w_ref, o_ref):
        dinv = lax.rsqrt(dp0_ref[0][:, :1] + dp1_ref[0][:, :1] + 1.0)
        s = s0_ref[0] + s1_ref[0] + g_ref[...]
        x2 = jnp.maximum(dinv * s + b_ref[...], 0.0)
        h2 = jnp.dot(x2, w_ref[...], preferred_element_type=jnp.float32)
        o_ref[...] = h2 * dinv

    return pl.pallas_call(
        body,
        grid=(n // _ROWS,),
        in_specs=_sp_specs(dh) + [
            pl.BlockSpec((_ROWS, dh), lambda i: (i, 0)),
        ] + _deg_specs() + [
            pl.BlockSpec((1, dh), lambda i: (0, 0)),
            pl.BlockSpec((dh, dout), lambda i: (0, 0)),
        ],
        out_specs=pl.BlockSpec((_ROWS, dout), lambda i: (i, 0)),
        out_shape=jax.ShapeDtypeStruct((n, dout), jnp.float32),
    )(sp, sp, g, degp, degp, b, w)


def _tc_combine(sp, g, degp, b):
    n, dh = g.shape

    def body(s0_ref, s1_ref, g_ref, dp0_ref, dp1_ref, b_ref, o_ref):
        dinv = lax.rsqrt(dp0_ref[0][:, :1] + dp1_ref[0][:, :1] + 1.0)
        s = s0_ref[0] + s1_ref[0] + g_ref[...]
        o_ref[...] = jnp.maximum(dinv * s + b_ref[...], 0.0)

    return pl.pallas_call(
        body,
        grid=(n // _ROWS,),
        in_specs=_sp_specs(dh) + [
            pl.BlockSpec((_ROWS, dh), lambda i: (i, 0)),
        ] + _deg_specs() + [
            pl.BlockSpec((1, dh), lambda i: (0, 0)),
        ],
        out_specs=pl.BlockSpec((_ROWS, dh), lambda i: (i, 0)),
        out_shape=jax.ShapeDtypeStruct((n, dh), jnp.float32),
    )(sp, sp, g, degp, degp, b)


def kernel(x, edge_index, W1, b1, W2, b2):
    n, _ = x.shape
    e = edge_index.shape[1]
    dh = W1.shape[1]
    rps = -(-((n + 1) // NS + 7) // 8) * 8
    npad = NS * rps

    c_msg = -(-(-(-e // (NW * KM))) // NB) * NB
    epad = NW * c_msg * KM - e
    pad_i = jnp.arange(epad, dtype=jnp.int32)
    src3 = jnp.concatenate(
        [edge_index[0], pad_i % n]).reshape(NW, c_msg, KM)
    dst3 = jnp.concatenate(
        [edge_index[1], n + pad_i % (npad - n)]).reshape(NW, c_msg, KM)

    epw = c_msg * KM
    kd = max(k for k in range(8, 129, 8) if epw % k == 0)
    dst3d = dst3.reshape(NW, epw // kd, kd)

    ones_rows = jnp.ones((kd, 16), jnp.float32)
    zeros16 = jnp.zeros((npad, 16), jnp.float32)
    zerosN = jnp.zeros((npad, dh), jnp.float32)
    b1r = b1.reshape(1, -1)
    b2r = b2.reshape(1, -1)

    degp = _sc_degree(dst3d, ones_rows, zeros16, npad)

    g1 = _tc_scale_matmul(x, W1, degp)
    sp1 = _sc_messages(g1, src3, dst3, zerosN, npad)
    g2 = _tc_combine_matmul(sp1, g1, degp, b1r, W2)
    sp2 = _sc_messages(g2, src3, dst3, zerosN, npad)
    return _tc_combine(sp2, g2, degp, b2r)
